# baseline (speedup 1.0000x reference)
import jax
import jax.numpy as jnp
import numpy as np
from jax.experimental import pallas as pl
from jax.experimental.pallas import tpu as pltpu

_RATE = 0.1
_SCALE = float(np.float32(1.0) / np.float32(1.0 - _RATE))
_THRESH = 838861
_K0 = 0
_K1 = 42
_KS2 = (_K0 ^ _K1 ^ 0x1BD11BDA)

_ROT1 = (13, 15, 26, 6)
_ROT2 = (17, 29, 16, 24)

_BLK_R2 = 512
_BLK_C2 = 128
_BLK = _BLK_R2 * _BLK_C2


def _rotl(x, d):
    return jax.lax.shift_left(x, jnp.int32(d)) | jax.lax.shift_right_logical(
        x, jnp.int32(32 - d)
    )


def _mix(x0, x1, rots):
    for r in rots:
        x0 = x0 + x1
        x1 = _rotl(x1, r) ^ x0
    return x0, x1


def _threefry_bits(p):
    ks0 = jnp.int32(_K0)
    ks1 = jnp.int32(_K1)
    ks2 = jnp.int32(_KS2)
    x0 = jnp.zeros_like(p) + ks0
    x1 = p + ks1
    x0, x1 = _mix(x0, x1, _ROT1)
    x0, x1 = x0 + ks1, x1 + (ks2 + jnp.int32(1))
    x0, x1 = _mix(x0, x1, _ROT2)
    x0, x1 = x0 + ks2, x1 + (ks0 + jnp.int32(2))
    x0, x1 = _mix(x0, x1, _ROT1)
    x0, x1 = x0 + ks0, x1 + (ks1 + jnp.int32(3))
    x0, x1 = _mix(x0, x1, _ROT2)
    x0, x1 = x0 + ks1, x1 + (ks2 + jnp.int32(4))
    x0, x1 = _mix(x0, x1, _ROT1)
    x0, x1 = x0 + ks2, x1 + (ks0 + jnp.int32(5))
    return x0 ^ x1


def _body(nblk, v_ref, o_ref):
    b = pl.program_id(0)
    row = jax.lax.broadcasted_iota(jnp.int32, (_BLK_R2, _BLK_C2), 0)
    col = jax.lax.broadcasted_iota(jnp.int32, (_BLK_R2, _BLK_C2), 1)
    p = b * _BLK + row * _BLK_C2 + col
    bits = _threefry_bits(p)
    keep = jax.lax.shift_right_logical(bits, jnp.int32(9)) >= jnp.int32(_THRESH)
    v = v_ref[...].reshape(_BLK_R2, _BLK_C2)
    res = jnp.where(keep, v * _SCALE, jnp.float32(0.0))
    o_ref[...] = res.reshape(_BLK)


def _tc_dropout(values, head, full_size=None, idx_flat=None):
    nblk = (head + _BLK - 1) // _BLK

    if idx_flat is None:
        def body(v_ref, o_ref):
            _body(nblk, v_ref, o_ref)

        return pl.pallas_call(
            body,
            grid=(nblk,),
            in_specs=[pl.BlockSpec((_BLK,), lambda b: (b,))],
            out_specs=pl.BlockSpec((_BLK,), lambda b: (b,)),
            out_shape=jax.ShapeDtypeStruct((full_size or head,), jnp.float32),
        )(values)

    ni = idx_flat.shape[0]
    iblk_r = (ni + nblk * 1024 - 1) // (nblk * 1024)
    iblk = iblk_r * 1024

    def body(v_ref, i_ref, o_ref, io_ref):
        _body(nblk, v_ref, o_ref)
        io_ref[...] = i_ref[...]

    return pl.pallas_call(
        body,
        grid=(nblk,),
        in_specs=[
            pl.BlockSpec((_BLK,), lambda b: (b,)),
            pl.BlockSpec((iblk,), lambda b: (b,)),
        ],
        out_specs=[
            pl.BlockSpec((_BLK,), lambda b: (b,)),
            pl.BlockSpec((iblk,), lambda b: (b,)),
        ],
        out_shape=[
            jax.ShapeDtypeStruct((full_size or head,), jnp.float32),
            jax.ShapeDtypeStruct((ni,), idx_flat.dtype),
        ],
    )(values, idx_flat)


from jax import lax
from jax.experimental.pallas import tpu_sc as plsc

_SC_CH = 4096
_SC_NW = 32
_SC_V = _SC_CH // 16


def _sc_dropout(values, start, count):
    nfull = count // _SC_CH
    tail_base = nfull * _SC_CH
    tail_len = count - tail_base
    tail_wid = nfull % _SC_NW
    tail_v = (tail_len + 15) // 16

    mesh = plsc.VectorSubcoreMesh(core_axis_name="c", subcore_axis_name="s")

    def body(v_hbm, o_hbm, vin, vout):
        c = lax.axis_index("c")
        s = lax.axis_index("s")
        wid = s * 2 + c

        def compute_chunk(gbase, nv):
            @plsc.parallel_loop(0, nv, unroll=8)
            def _(j):
                p = gbase + j * 16 + lax.iota(jnp.int32, 16)
                bits = _threefry_bits(p)
                keep = (
                    jax.lax.shift_right_logical(bits, jnp.int32(9))
                    >= jnp.int32(_THRESH)
                )
                x = vin[pl.ds(j * 16, 16)]
                vout[pl.ds(j * 16, 16)] = jnp.where(
                    keep, x * _SCALE, jnp.float32(0.0)
                )

        nt = (jnp.int32(nfull) - wid + jnp.int32(_SC_NW - 1)) // jnp.int32(_SC_NW)

        def one_chunk(t, _):
            lbase = (wid + t * _SC_NW) * _SC_CH
            pltpu.sync_copy(v_hbm.at[pl.ds(start + lbase, _SC_CH)], vin)
            compute_chunk(start + lbase, _SC_V)
            pltpu.sync_copy(vout, o_hbm.at[pl.ds(lbase, _SC_CH)])
            return 0

        lax.fori_loop(0, nt, one_chunk, 0)

        if tail_len:
            @pl.when(wid == tail_wid)
            def _tail():
                pltpu.sync_copy(
                    v_hbm.at[pl.ds(start + tail_base, tail_len)],
                    vin.at[pl.ds(0, tail_len)],
                )
                compute_chunk(jnp.int32(start + tail_base), tail_v)
                pltpu.sync_copy(
                    vout.at[pl.ds(0, tail_len)],
                    o_hbm.at[pl.ds(tail_base, tail_len)],
                )

    return pl.kernel(
        body,
        out_type=jax.ShapeDtypeStruct((count,), jnp.float32),
        mesh=mesh,
        scratch_types=[
            pltpu.VMEM((_SC_CH,), jnp.float32),
            pltpu.VMEM((_SC_CH,), jnp.float32),
        ],
    )(values)


_SPLIT = 32 * _BLK


def _tc_writeback(tmp, full):
    count = tmp.shape[0]
    nblk = (count + _BLK - 1) // _BLK
    off = _SPLIT // _BLK

    def body(t_ref, f_ref, o_ref):
        o_ref[...] = t_ref[...]

    return pl.pallas_call(
        body,
        grid=(nblk,),
        in_specs=[
            pl.BlockSpec((_BLK,), lambda b: (b,)),
            pl.BlockSpec(memory_space=pl.ANY),
        ],
        out_specs=pl.BlockSpec((_BLK,), lambda b: (b + off,)),
        out_shape=jax.ShapeDtypeStruct(full.shape, jnp.float32),
        input_output_aliases={1: 0},
    )(tmp, full)


def _tc_dropout_idx64(values, idx64):
    nnz = values.shape[0]
    nblk = (nnz + _BLK - 1) // _BLK

    def body(v_ref, i64_ref, o_ref, io64_ref, sem):
        b = pl.program_id(0)

        @pl.when(b == 0)
        def _start():
            pltpu.make_async_copy(i64_ref, io64_ref, sem).start()

        _body(nblk, v_ref, o_ref)

        @pl.when(b == nblk - 1)
        def _wait():
            pltpu.make_async_copy(i64_ref, io64_ref, sem).wait()

    return pl.pallas_call(
        body,
        grid=(nblk,),
        in_specs=[
            pl.BlockSpec((_BLK,), lambda b: (b,)),
            pl.BlockSpec(memory_space=pl.ANY),
        ],
        out_specs=[
            pl.BlockSpec((_BLK,), lambda b: (b,)),
            pl.BlockSpec(memory_space=pl.ANY),
        ],
        out_shape=[
            jax.ShapeDtypeStruct((nnz,), jnp.float32),
            jax.ShapeDtypeStruct(idx64.shape, idx64.dtype),
        ],
        scratch_shapes=[pltpu.SemaphoreType.DMA],
    )(values, idx64)


def kernel(values, indices):
    idx64 = jax.lax.bitcast_convert_type(indices, jnp.int64)
    out, idx_out64 = _tc_dropout_idx64(values, idx64)
    return out, jax.lax.bitcast_convert_type(idx_out64, jnp.int32)


# speedup vs baseline: 559.3980x; 559.3980x over previous
import jax
import jax.numpy as jnp
import numpy as np
from jax.experimental import pallas as pl
from jax.experimental.pallas import tpu as pltpu

_RATE = 0.1
_SCALE = float(np.float32(1.0) / np.float32(1.0 - _RATE))
_THRESH = 838861
_K0 = 0
_K1 = 42
_KS2 = (_K0 ^ _K1 ^ 0x1BD11BDA)

_ROT1 = (13, 15, 26, 6)
_ROT2 = (17, 29, 16, 24)

_BLK_R2 = 512
_BLK_C2 = 128
_BLK = _BLK_R2 * _BLK_C2


def _rotl(x, d):
    return jax.lax.shift_left(x, jnp.int32(d)) | jax.lax.shift_right_logical(
        x, jnp.int32(32 - d)
    )


def _mix(x0, x1, rots):
    for r in rots:
        x0 = x0 + x1
        x1 = _rotl(x1, r) ^ x0
    return x0, x1


def _threefry_bits(p):
    ks0 = jnp.int32(_K0)
    ks1 = jnp.int32(_K1)
    ks2 = jnp.int32(_KS2)
    x0 = jnp.zeros_like(p) + ks0
    x1 = p + ks1
    x0, x1 = _mix(x0, x1, _ROT1)
    x0, x1 = x0 + ks1, x1 + (ks2 + jnp.int32(1))
    x0, x1 = _mix(x0, x1, _ROT2)
    x0, x1 = x0 + ks2, x1 + (ks0 + jnp.int32(2))
    x0, x1 = _mix(x0, x1, _ROT1)
    x0, x1 = x0 + ks0, x1 + (ks1 + jnp.int32(3))
    x0, x1 = _mix(x0, x1, _ROT2)
    x0, x1 = x0 + ks1, x1 + (ks2 + jnp.int32(4))
    x0, x1 = _mix(x0, x1, _ROT1)
    x0, x1 = x0 + ks2, x1 + (ks0 + jnp.int32(5))
    return x0 ^ x1


def _body(nblk, v_ref, o_ref):
    b = pl.program_id(0)
    row = jax.lax.broadcasted_iota(jnp.int32, (_BLK_R2, _BLK_C2), 0)
    col = jax.lax.broadcasted_iota(jnp.int32, (_BLK_R2, _BLK_C2), 1)
    p = b * _BLK + row * _BLK_C2 + col
    bits = _threefry_bits(p)
    keep = jax.lax.shift_right_logical(bits, jnp.int32(9)) >= jnp.int32(_THRESH)
    v = v_ref[...].reshape(_BLK_R2, _BLK_C2)
    res = jnp.where(keep, v * _SCALE, jnp.float32(0.0))
    o_ref[...] = res.reshape(_BLK)


def _tc_dropout(values, head, full_size=None, idx_flat=None):
    nblk = (head + _BLK - 1) // _BLK

    if idx_flat is None:
        def body(v_ref, o_ref):
            _body(nblk, v_ref, o_ref)

        return pl.pallas_call(
            body,
            grid=(nblk,),
            in_specs=[pl.BlockSpec((_BLK,), lambda b: (b,))],
            out_specs=pl.BlockSpec((_BLK,), lambda b: (b,)),
            out_shape=jax.ShapeDtypeStruct((full_size or head,), jnp.float32),
        )(values)

    ni = idx_flat.shape[0]
    iblk_r = (ni + nblk * 1024 - 1) // (nblk * 1024)
    iblk = iblk_r * 1024

    def body(v_ref, i_ref, o_ref, io_ref):
        _body(nblk, v_ref, o_ref)
        io_ref[...] = i_ref[...]

    return pl.pallas_call(
        body,
        grid=(nblk,),
        in_specs=[
            pl.BlockSpec((_BLK,), lambda b: (b,)),
            pl.BlockSpec((iblk,), lambda b: (b,)),
        ],
        out_specs=[
            pl.BlockSpec((_BLK,), lambda b: (b,)),
            pl.BlockSpec((iblk,), lambda b: (b,)),
        ],
        out_shape=[
            jax.ShapeDtypeStruct((full_size or head,), jnp.float32),
            jax.ShapeDtypeStruct((ni,), idx_flat.dtype),
        ],
    )(values, idx_flat)


from jax import lax
from jax.experimental.pallas import tpu_sc as plsc

_SC_CH = 4096
_SC_NW = 32
_SC_V = _SC_CH // 16


def _sc_dropout(values, start, count):
    nfull = count // _SC_CH
    tail_base = nfull * _SC_CH
    tail_len = count - tail_base
    tail_wid = nfull % _SC_NW
    tail_v = (tail_len + 15) // 16

    mesh = plsc.VectorSubcoreMesh(core_axis_name="c", subcore_axis_name="s")

    def body(v_hbm, o_hbm, vin, vout):
        c = lax.axis_index("c")
        s = lax.axis_index("s")
        wid = s * 2 + c

        def compute_chunk(gbase, nv):
            @plsc.parallel_loop(0, nv, unroll=8)
            def _(j):
                p = gbase + j * 16 + lax.iota(jnp.int32, 16)
                bits = _threefry_bits(p)
                keep = (
                    jax.lax.shift_right_logical(bits, jnp.int32(9))
                    >= jnp.int32(_THRESH)
                )
                x = vin[pl.ds(j * 16, 16)]
                vout[pl.ds(j * 16, 16)] = jnp.where(
                    keep, x * _SCALE, jnp.float32(0.0)
                )

        nt = (jnp.int32(nfull) - wid + jnp.int32(_SC_NW - 1)) // jnp.int32(_SC_NW)

        def one_chunk(t, _):
            lbase = (wid + t * _SC_NW) * _SC_CH
            pltpu.sync_copy(v_hbm.at[pl.ds(start + lbase, _SC_CH)], vin)
            compute_chunk(start + lbase, _SC_V)
            pltpu.sync_copy(vout, o_hbm.at[pl.ds(lbase, _SC_CH)])
            return 0

        lax.fori_loop(0, nt, one_chunk, 0)

        if tail_len:
            @pl.when(wid == tail_wid)
            def _tail():
                pltpu.sync_copy(
                    v_hbm.at[pl.ds(start + tail_base, tail_len)],
                    vin.at[pl.ds(0, tail_len)],
                )
                compute_chunk(jnp.int32(start + tail_base), tail_v)
                pltpu.sync_copy(
                    vout.at[pl.ds(0, tail_len)],
                    o_hbm.at[pl.ds(tail_base, tail_len)],
                )

    return pl.kernel(
        body,
        out_type=jax.ShapeDtypeStruct((count,), jnp.float32),
        mesh=mesh,
        scratch_types=[
            pltpu.VMEM((_SC_CH,), jnp.float32),
            pltpu.VMEM((_SC_CH,), jnp.float32),
        ],
    )(values)


_SPLIT = 32 * _BLK


def _tc_writeback(tmp, full):
    count = tmp.shape[0]
    nblk = (count + _BLK - 1) // _BLK
    off = _SPLIT // _BLK

    def body(t_ref, f_ref, o_ref):
        o_ref[...] = t_ref[...]

    return pl.pallas_call(
        body,
        grid=(nblk,),
        in_specs=[
            pl.BlockSpec((_BLK,), lambda b: (b,)),
            pl.BlockSpec(memory_space=pl.ANY),
        ],
        out_specs=pl.BlockSpec((_BLK,), lambda b: (b + off,)),
        out_shape=jax.ShapeDtypeStruct(full.shape, jnp.float32),
        input_output_aliases={1: 0},
    )(tmp, full)


def kernel(values, indices):
    nnz = values.shape[0]
    tail = _sc_dropout(values, _SPLIT, nnz - _SPLIT)
    head_full = _tc_dropout(values, _SPLIT, full_size=nnz)
    out = _tc_writeback(tail, head_full)
    return out, indices
